# baseline (speedup 1.0000x reference)
import jax
import jax.numpy as jnp
from jax.experimental import pallas as pl
from jax.experimental.pallas import tpu as pltpu

_BM = 512


def _body(a_ref, o_ref):
    a = a_ref[...]
    q = jnp.clip(jnp.abs(4.0 * a) - 3.0, 0.0, 30.0)
    q = q * q
    t = a * a
    ii = jax.lax.broadcasted_iota(jnp.int32, (256, 128), 0)
    jj = jax.lax.broadcasted_iota(jnp.int32, (256, 128), 1)
    e = ((ii // 2) == jj).astype(jnp.float32)
    s = jax.lax.dot(t, e, preferred_element_type=jnp.float32)
    qs = jax.lax.dot(q, e, preferred_element_type=jnp.float32)
    jerk = jnp.clip(8.0 * jnp.sqrt(s) - 1.0, 0.0, 20.0)
    o_ref[...] = 0.1 * (qs + jerk * jerk)


def kernel(trajs):
    b, n, _ = trajs.shape
    total = b * n
    rows = total // 128
    flat = trajs.reshape(rows, 256)
    out = pl.pallas_call(
        _body,
        grid=(rows // _BM,),
        in_specs=[pl.BlockSpec((_BM, 256), lambda i: (i, 0))],
        out_specs=pl.BlockSpec((_BM, 128), lambda i: (i, 0)),
        out_shape=jax.ShapeDtypeStruct((rows, 128), jnp.float32),
    )(flat)
    return out.reshape(b, n)


# speedup vs baseline: 12.4847x; 12.4847x over previous
import jax
import jax.numpy as jnp
from jax.experimental import pallas as pl
from jax.experimental.pallas import tpu as pltpu

_BM = 512


_BN = 256


def _body(a_ref, o_ref):
    a = a_ref[...]
    q = jnp.clip(jnp.abs(4.0 * a) - 3.0, 0.0, 30.0)
    q = q * q
    t = a * a
    ii = jax.lax.broadcasted_iota(jnp.int32, (_BN, _BN // 2), 0)
    jj = jax.lax.broadcasted_iota(jnp.int32, (_BN, _BN // 2), 1)
    e = ((ii // 2) == jj).astype(jnp.float32)
    s = jax.lax.dot(t, e, preferred_element_type=jnp.float32)
    qs = jax.lax.dot(q, e, preferred_element_type=jnp.float32)
    jerk = jnp.clip(8.0 * jnp.sqrt(s) - 1.0, 0.0, 20.0)
    o_ref[...] = 0.1 * (qs + jerk * jerk)


def kernel(trajs):
    b, n, _ = trajs.shape
    flat = trajs.reshape(b, 2 * n)
    out = pl.pallas_call(
        _body,
        grid=(2 * n // _BN,),
        in_specs=[pl.BlockSpec((b, _BN), lambda i: (0, i))],
        out_specs=pl.BlockSpec((b, _BN // 2), lambda i: (0, i)),
        out_shape=jax.ShapeDtypeStruct((b, n), jnp.float32),
    )(flat)
    return out


# speedup vs baseline: 28.2013x; 2.2589x over previous
import jax
import jax.numpy as jnp
from jax.experimental import pallas as pl

_BN = 4096
_C = 256
_G = 4


def _body(a_ref, o_ref):
    a = a_ref[...]
    b = a.shape[0]
    q = jnp.clip(jnp.abs(4.0 * a) - 3.0, 0.0, 30.0)
    q = q * q
    t = a * a
    ii = jax.lax.broadcasted_iota(jnp.int32, (_C, _C // 2), 0)
    jj = jax.lax.broadcasted_iota(jnp.int32, (_C, _C // 2), 1)
    e = ((ii // 2) == jj).astype(jnp.float32)
    for m in range(_BN // (_C * _G)):
        qg = jnp.concatenate(
            [q[:, (m * _G + u) * _C:(m * _G + u + 1) * _C] for u in range(_G)], axis=0)
        tg = jnp.concatenate(
            [t[:, (m * _G + u) * _C:(m * _G + u + 1) * _C] for u in range(_G)], axis=0)
        qs = jax.lax.dot(qg, e, preferred_element_type=jnp.float32)
        s = jax.lax.dot(tg, e, preferred_element_type=jnp.float32)
        jerk = jnp.clip(8.0 * jnp.sqrt(s) - 1.0, 0.0, 20.0)
        res = 0.1 * (qs + jerk * jerk)
        for u in range(_G):
            o_ref[:, (m * _G + u) * (_C // 2):(m * _G + u + 1) * (_C // 2)] = (
                res[u * b:(u + 1) * b, :])


def kernel(trajs):
    b, n, _ = trajs.shape
    flat = trajs.reshape(b, 2 * n)
    out = pl.pallas_call(
        _body,
        grid=(2 * n // _BN,),
        in_specs=[pl.BlockSpec((b, _BN), lambda i: (0, i))],
        out_specs=pl.BlockSpec((b, _BN // 2), lambda i: (0, i)),
        out_shape=jax.ShapeDtypeStruct((b, n), jnp.float32),
    )(flat)
    return out


# speedup vs baseline: 142.9743x; 5.0698x over previous
import jax
import jax.numpy as jnp
from jax.experimental import pallas as pl

_BN = 2048


def _body(a_ref, o_ref):
    x = a_ref[:, 0, :]
    y = a_ref[:, 1, :]
    qx = jnp.clip(jnp.abs(4.0 * x) - 3.0, 0.0, 30.0)
    qy = jnp.clip(jnp.abs(4.0 * y) - 3.0, 0.0, 30.0)
    s = x * x + y * y
    jerk = jnp.clip(8.0 * jnp.sqrt(s) - 1.0, 0.0, 20.0)
    o_ref[...] = 0.1 * (qx * qx + qy * qy + jerk * jerk)


def kernel(trajs):
    b, n, _ = trajs.shape
    planes = jnp.transpose(trajs, (0, 2, 1))
    out = pl.pallas_call(
        _body,
        grid=(n // _BN,),
        in_specs=[pl.BlockSpec((b, 2, _BN), lambda i: (0, 0, i))],
        out_specs=pl.BlockSpec((b, _BN), lambda i: (0, i)),
        out_shape=jax.ShapeDtypeStruct((b, n), jnp.float32),
    )(planes)
    return out
